# baseline (speedup 1.0000x reference)
import functools

import jax
import jax.numpy as jnp
from jax import lax
from jax.experimental import pallas as pl
from jax.experimental.pallas import tpu as pltpu
from jax.experimental.pallas import tpu_sc as plsc

CLS = 100000
FEAT = 64
B = 16384

NSUB = 16
CHUNK = B // NSUB
HALF = CHUNK // 2
NROW = HALF // 128
CLS_PAD = 100352
STRIPE = CLS_PAD // NSUB


def _body(xs_hbm, ys_hbm, center_hbm, out_hbm,
          cnt_sh, dsum_sh, part_sh,
          idx_v, xs_v, crows, dvals, ones_v, stage, stage2, fin_v, sem):
    cid = lax.axis_index("c")
    sid = lax.axis_index("s")

    @pl.when(cid == 0)
    def _():
        zero16 = jnp.zeros((16,), jnp.float32)
        one16 = jnp.ones((16,), jnp.float32)

        def zfill(i, _):
            stage[pl.ds(i * 16, 16)] = zero16
            return 0
        lax.fori_loop(0, STRIPE // 16, zfill, 0)
        pltpu.sync_copy(stage.at[pl.ds(0, STRIPE)],
                        cnt_sh.at[pl.ds(sid * STRIPE, STRIPE)])
        pltpu.sync_copy(stage.at[pl.ds(0, STRIPE)],
                        dsum_sh.at[pl.ds(sid * STRIPE, STRIPE)])
        for j in range(NROW):
            for k in range(128 // 16):
                ones_v[j, pl.ds(k * 16, 16)] = one16

        plsc.subcore_barrier()

        for half in range(2):
            base = sid * CHUNK + half * HALF
            for j in range(NROW):
                pltpu.sync_copy(ys_hbm.at[pl.ds(base + j * 128, 128)],
                                idx_v.at[j])
            cps = [
                pltpu.async_copy(center_hbm.at[idx_v.at[j]],
                                 crows.at[pl.ds(j * 128, 128)], sem)
                for j in range(NROW)
            ]
            pltpu.sync_copy(xs_hbm.at[pl.ds(base, HALF)], xs_v)
            for cp in cps:
                cp.wait()

            for j in range(NROW):
                def dgroup(g, _, j=j):
                    e_idx = (j * 128 + g * 16
                             + lax.iota(jnp.int32, 16))
                    acc = jnp.zeros((16,), jnp.float32)
                    for f in range(FEAT):
                        f_idx = jnp.full((16,), f, jnp.int32)
                        xv = plsc.load_gather(xs_v, [e_idx, f_idx])
                        cv = plsc.load_gather(crows, [e_idx, f_idx])
                        df = xv - cv
                        acc = acc + df * df
                    dvals[j, pl.ds(g * 16, 16)] = acc
                    return 0
                lax.fori_loop(0, 128 // 16, dgroup, 0)

            for j in range(NROW):
                pltpu.sync_copy(ones_v.at[j], cnt_sh.at[idx_v.at[j]],
                                add=True)
                pltpu.sync_copy(dvals.at[j], dsum_sh.at[idx_v.at[j]],
                                add=True)

        plsc.subcore_barrier()

        pltpu.sync_copy(cnt_sh.at[pl.ds(sid * STRIPE, STRIPE)], stage)
        pltpu.sync_copy(dsum_sh.at[pl.ds(sid * STRIPE, STRIPE)], stage2)

        def rstep(i, acc):
            n = stage[pl.ds(i * 16, 16)]
            dv = stage2[pl.ds(i * 16, 16)]
            return acc + dv / (n + n + 2.0)
        accv = lax.fori_loop(0, STRIPE // 16, rstep,
                             jnp.zeros((16,), jnp.float32))
        fin_v[0, pl.ds(0, 16)] = accv
        pltpu.sync_copy(fin_v.at[0], part_sh.at[sid])

        plsc.subcore_barrier()

        @pl.when(sid == 0)
        def _():
            pltpu.sync_copy(part_sh, fin_v)
            tot = jnp.zeros((16,), jnp.float32)
            for r in range(NSUB):
                tot = tot + fin_v[r, pl.ds(0, 16)]
            tot = plsc.cumsum(tot) * (1.0 / B)
            fin_v[0, pl.ds(0, 16)] = tot
            pltpu.sync_copy(fin_v.at[0], out_hbm)


@jax.jit
def _center_loss(xs, ys, center):
    kern = pl.kernel(
        _body,
        out_type=jax.ShapeDtypeStruct((16,), jnp.float32),
        mesh=plsc.VectorSubcoreMesh(core_axis_name="c", subcore_axis_name="s"),
        compiler_params=pltpu.CompilerParams(
            needs_layout_passes=False, use_tc_tiling_on_sc=False),
        scratch_types=[
            pltpu.VMEM_SHARED((CLS_PAD,), jnp.float32),
            pltpu.VMEM_SHARED((CLS_PAD,), jnp.float32),
            pltpu.VMEM_SHARED((NSUB, 16), jnp.float32),
            pltpu.VMEM((NROW, 128), jnp.int32),
            pltpu.VMEM((HALF, FEAT), jnp.float32),
            pltpu.VMEM((HALF, FEAT), jnp.float32),
            pltpu.VMEM((NROW, 128), jnp.float32),
            pltpu.VMEM((NROW, 128), jnp.float32),
            pltpu.VMEM((STRIPE,), jnp.float32),
            pltpu.VMEM((STRIPE,), jnp.float32),
            pltpu.VMEM((NSUB, 16), jnp.float32),
            pltpu.SemaphoreType.DMA,
        ],
    )
    return kern(xs, ys, center)


def kernel(xs, ys, center):
    out = _center_loss(xs, ys.astype(jnp.int32), center)
    return out[15]


# speedup vs baseline: 1.4242x; 1.4242x over previous
import functools

import jax
import jax.numpy as jnp
from jax import lax
from jax.experimental import pallas as pl
from jax.experimental.pallas import tpu as pltpu
from jax.experimental.pallas import tpu_sc as plsc

CLS = 100000
FEAT = 64
B = 16384

NSUB = 16
CHUNK = B // NSUB
HALF = CHUNK // 2
NROW = HALF // 128
CLS_PAD = 100352
STRIPE = CLS_PAD // NSUB


def _body(xs_hbm, ys_hbm, center_hbm, out_hbm,
          cnt_sh, dsum_sh, part_sh,
          idx_v, xs_v, crows, dvals, ones_v, stage, stage2, fin_v, sem):
    cid = lax.axis_index("c")
    sid = lax.axis_index("s")

    @pl.when(cid == 0)
    def _():
        zero16 = jnp.zeros((16,), jnp.float32)
        one16 = jnp.ones((16,), jnp.float32)

        def zfill(i, _):
            stage[pl.ds(i * 16, 16)] = zero16
            return 0
        lax.fori_loop(0, STRIPE // 16, zfill, 0)
        pltpu.sync_copy(stage.at[pl.ds(0, STRIPE)],
                        cnt_sh.at[pl.ds(sid * STRIPE, STRIPE)])
        pltpu.sync_copy(stage.at[pl.ds(0, STRIPE)],
                        dsum_sh.at[pl.ds(sid * STRIPE, STRIPE)])
        for j in range(NROW):
            for k in range(128 // 16):
                ones_v[j, pl.ds(k * 16, 16)] = one16

        plsc.subcore_barrier()

        for half in range(2):
            base = sid * CHUNK + half * HALF
            for j in range(NROW):
                pltpu.sync_copy(ys_hbm.at[pl.ds(base + j * 128, 128)],
                                idx_v.at[j])
            cps = [
                pltpu.async_copy(center_hbm.at[idx_v.at[j]],
                                 crows.at[pl.ds(j * 128, 128)], sem)
                for j in range(NROW)
            ]
            pltpu.sync_copy(xs_hbm.at[pl.ds(base, HALF)], xs_v)
            for cp in cps:
                cp.wait()

            for j in range(NROW):
                def dgroup(g, _, j=j):
                    lane = lax.iota(jnp.int32, 16)
                    e_idx = j * 128 + g * 16 + lane
                    acc = jnp.zeros((16,), jnp.float32)
                    for f0 in range(FEAT):
                        f_idx = (lane + f0) & (FEAT - 1)
                        xv = plsc.load_gather(xs_v, [e_idx, f_idx])
                        cv = plsc.load_gather(crows, [e_idx, f_idx])
                        df = xv - cv
                        acc = acc + df * df
                    dvals[j, pl.ds(g * 16, 16)] = acc
                    return 0
                lax.fori_loop(0, 128 // 16, dgroup, 0)

            for j in range(NROW):
                pltpu.sync_copy(ones_v.at[j], cnt_sh.at[idx_v.at[j]],
                                add=True)
                pltpu.sync_copy(dvals.at[j], dsum_sh.at[idx_v.at[j]],
                                add=True)

        plsc.subcore_barrier()

        pltpu.sync_copy(cnt_sh.at[pl.ds(sid * STRIPE, STRIPE)], stage)
        pltpu.sync_copy(dsum_sh.at[pl.ds(sid * STRIPE, STRIPE)], stage2)

        def rstep(i, acc):
            n = stage[pl.ds(i * 16, 16)]
            dv = stage2[pl.ds(i * 16, 16)]
            return acc + dv / (n + n + 2.0)
        accv = lax.fori_loop(0, STRIPE // 16, rstep,
                             jnp.zeros((16,), jnp.float32))
        fin_v[0, pl.ds(0, 16)] = accv
        pltpu.sync_copy(fin_v.at[0], part_sh.at[sid])

        plsc.subcore_barrier()

        @pl.when(sid == 0)
        def _():
            pltpu.sync_copy(part_sh, fin_v)
            tot = jnp.zeros((16,), jnp.float32)
            for r in range(NSUB):
                tot = tot + fin_v[r, pl.ds(0, 16)]
            tot = plsc.cumsum(tot) * (1.0 / B)
            fin_v[0, pl.ds(0, 16)] = tot
            pltpu.sync_copy(fin_v.at[0], out_hbm)


@jax.jit
def _center_loss(xs, ys, center):
    kern = pl.kernel(
        _body,
        out_type=jax.ShapeDtypeStruct((16,), jnp.float32),
        mesh=plsc.VectorSubcoreMesh(core_axis_name="c", subcore_axis_name="s"),
        compiler_params=pltpu.CompilerParams(
            needs_layout_passes=False, use_tc_tiling_on_sc=False),
        scratch_types=[
            pltpu.VMEM_SHARED((CLS_PAD,), jnp.float32),
            pltpu.VMEM_SHARED((CLS_PAD,), jnp.float32),
            pltpu.VMEM_SHARED((NSUB, 16), jnp.float32),
            pltpu.VMEM((NROW, 128), jnp.int32),
            pltpu.VMEM((HALF, FEAT), jnp.float32),
            pltpu.VMEM((HALF, FEAT), jnp.float32),
            pltpu.VMEM((NROW, 128), jnp.float32),
            pltpu.VMEM((NROW, 128), jnp.float32),
            pltpu.VMEM((STRIPE,), jnp.float32),
            pltpu.VMEM((STRIPE,), jnp.float32),
            pltpu.VMEM((NSUB, 16), jnp.float32),
            pltpu.SemaphoreType.DMA,
        ],
    )
    return kern(xs, ys, center)


def kernel(xs, ys, center):
    out = _center_loss(xs, ys.astype(jnp.int32), center)
    return out[15]
